# Initial kernel scaffold; baseline (speedup 1.0000x reference)
import jax
import jax.numpy as jnp
from jax import lax
from jax.experimental import pallas as pl
from jax.experimental.pallas import tpu as pltpu
from jax.experimental.pallas import tpu_sc as plsc

NC = 2
NS = 16
NW = NC * NS
L = 16

_B = 8192
_T = 16
_ML = 543
_SUB = 64
_BPW = _B // NW
_NSUB = _BPW // _SUB
_STARTS = tuple(range(0, _ML - L + 1, L)) + ((_ML - L),)


def _sc_body(inp_hbm, tid_hbm, off_hbm, disc_hbm, mask_hbm, out_hbm,
             tid_v, inp_v, off_v, disc_v, mask_v, a_v, c_v, out_v):
    wid = lax.axis_index("s") * NC + lax.axis_index("c")
    base = wid * _BPW

    pltpu.sync_copy(tid_hbm.at[pl.ds(base, _BPW)], tid_v)
    pltpu.sync_copy(inp_hbm.at[pl.ds(base, _BPW)], inp_v)
    pltpu.sync_copy(off_hbm, off_v)
    pltpu.sync_copy(disc_hbm, disc_v)
    pltpu.sync_copy(mask_hbm, mask_v)

    def fuse_row(t, _):
        for st in _STARTS:
            d = disc_v[t, pl.ds(st, L)]
            m = mask_v[t, pl.ds(st, L)]
            o = off_v[t, pl.ds(st, L)]
            a = d * m
            a_v[t, pl.ds(st, L)] = a
            c_v[t, pl.ds(st, L)] = a * o
        return 0
    lax.fori_loop(0, _T, fuse_row, 0)

    for sub in range(_NSUB):
        def row_body(r, _):
            t = tid_v[sub * _SUB + r]
            s = inp_v[sub * _SUB + r]
            for st in _STARTS:
                a = a_v[t, pl.ds(st, L)]
                c = c_v[t, pl.ds(st, L)]
                out_v[r, pl.ds(st, L)] = a * s + c
            return 0
        lax.fori_loop(0, _SUB, row_body, 0)
        pltpu.sync_copy(out_v, out_hbm.at[pl.ds(base + sub * _SUB, _SUB)])


@jax.jit
def _sc_affine(inp1, task_ids, offsets, discrimination, mask):
    kfn = pl.kernel(
        _sc_body,
        out_type=jax.ShapeDtypeStruct((_B, _ML), jnp.float32),
        mesh=plsc.VectorSubcoreMesh(core_axis_name="c", subcore_axis_name="s"),
        scratch_types=[
            pltpu.VMEM((_BPW,), jnp.int32),
            pltpu.VMEM((_BPW,), jnp.float32),
            pltpu.VMEM((_T, _ML), jnp.float32),
            pltpu.VMEM((_T, _ML), jnp.float32),
            pltpu.VMEM((_T, _ML), jnp.float32),
            pltpu.VMEM((_T, _ML), jnp.float32),
            pltpu.VMEM((_T, _ML), jnp.float32),
            pltpu.VMEM((_SUB, _ML), jnp.float32),
        ],
    )
    return kfn(inp1, task_ids, offsets, discrimination, mask)


def kernel(inp, task_ids, offsets, discrimination, mask):
    return _sc_affine(inp.reshape(-1), task_ids, offsets, discrimination, mask)


# speedup vs baseline: 1.1580x; 1.1580x over previous
import jax
import jax.numpy as jnp
from jax import lax
from jax.experimental import pallas as pl
from jax.experimental.pallas import tpu as pltpu
from jax.experimental.pallas import tpu_sc as plsc

NC = 2
NS = 16
NW = NC * NS
L = 16

_B = 8192
_T = 16
_ML = 543
_SUB = 64
_BPW = _B // NW
_NSUB = _BPW // _SUB
_STARTS = tuple(range(0, _ML - L + 1, L)) + ((_ML - L),)


def _sc_body(inp_hbm, tid_hbm, off_hbm, disc_hbm, mask_hbm, out_hbm,
             tid_v, inp_v, off_v, disc_v, mask_v, a_v, c_v, out_v):
    wid = lax.axis_index("s") * NC + lax.axis_index("c")
    base = wid * _BPW

    pltpu.sync_copy(tid_hbm.at[pl.ds(base, _BPW)], tid_v)
    pltpu.sync_copy(inp_hbm.at[pl.ds(base, _BPW)], inp_v)
    pltpu.sync_copy(off_hbm, off_v)
    pltpu.sync_copy(disc_hbm, disc_v)
    pltpu.sync_copy(mask_hbm, mask_v)

    def fuse_row(t, _):
        for st in _STARTS:
            d = disc_v[t, pl.ds(st, L)]
            m = mask_v[t, pl.ds(st, L)]
            o = off_v[t, pl.ds(st, L)]
            a = d * m
            a_v[t, pl.ds(st, L)] = a
            c_v[t, pl.ds(st, L)] = a * o
        return 0
    lax.fori_loop(0, _T, fuse_row, 0)

    def grp_body(g, _):
        tid16 = tid_v[pl.ds(g * L, L)]
        inp16 = inp_v[pl.ds(g * L, L)]
        vrow = lax.rem(g, _SUB // L) * L
        for k in range(L):
            t = tid16[k]
            s = inp16[k]
            for st in _STARTS:
                a = a_v[t, pl.ds(st, L)]
                c = c_v[t, pl.ds(st, L)]
                out_v[vrow + k, pl.ds(st, L)] = a * s + c

        @pl.when(lax.rem(g, _SUB // L) == _SUB // L - 1)
        def _flush():
            chunk = lax.div(g, _SUB // L)
            pltpu.sync_copy(out_v, out_hbm.at[pl.ds(base + chunk * _SUB, _SUB)])
        return 0
    lax.fori_loop(0, _BPW // L, grp_body, 0)


@jax.jit
def _sc_affine(inp1, task_ids, offsets, discrimination, mask):
    kfn = pl.kernel(
        _sc_body,
        out_type=jax.ShapeDtypeStruct((_B, _ML), jnp.float32),
        mesh=plsc.VectorSubcoreMesh(core_axis_name="c", subcore_axis_name="s"),
        scratch_types=[
            pltpu.VMEM((_BPW,), jnp.int32),
            pltpu.VMEM((_BPW,), jnp.float32),
            pltpu.VMEM((_T, _ML), jnp.float32),
            pltpu.VMEM((_T, _ML), jnp.float32),
            pltpu.VMEM((_T, _ML), jnp.float32),
            pltpu.VMEM((_T, _ML), jnp.float32),
            pltpu.VMEM((_T, _ML), jnp.float32),
            pltpu.VMEM((_SUB, _ML), jnp.float32),
        ],
    )
    return kfn(inp1, task_ids, offsets, discrimination, mask)


def kernel(inp, task_ids, offsets, discrimination, mask):
    return _sc_affine(inp.reshape(-1), task_ids, offsets, discrimination, mask)


# speedup vs baseline: 1.6744x; 1.4459x over previous
import jax
import jax.numpy as jnp
from jax import lax
from jax.experimental import pallas as pl
from jax.experimental.pallas import tpu as pltpu
from jax.experimental.pallas import tpu_sc as plsc

NC = 2
NS = 16
NW = NC * NS
L = 16

_B = 8192
_T = 16
_ML = 543
_BPW = _B // NW
_NGRP = _BPW // L
_STARTS = tuple(range(0, _ML - L + 1, L)) + ((_ML - L),)
_CH = 4


def _chunks(seq, n):
    return [seq[i:i + n] for i in range(0, len(seq), n)]


def _sc_body(inp_hbm, tid_hbm, off_hbm, disc_hbm, mask_hbm, out_hbm,
             tid_v, inp_v, off_v, disc_v, mask_v, a_v, c_v, out_v, sem):
    wid = lax.axis_index("s") * NC + lax.axis_index("c")
    base = wid * _BPW

    cps = [
        pltpu.async_copy(tid_hbm.at[pl.ds(base, _BPW)], tid_v, sem),
        pltpu.async_copy(inp_hbm.at[pl.ds(base, _BPW)], inp_v, sem),
        pltpu.async_copy(off_hbm, off_v, sem),
        pltpu.async_copy(disc_hbm, disc_v, sem),
        pltpu.async_copy(mask_hbm, mask_v, sem),
    ]
    for cp in cps:
        cp.wait()

    def fuse_row(t, _):
        for blks in _chunks(_STARTS, _CH):
            ds_ = [disc_v[t, pl.ds(st, L)] for st in blks]
            ms = [mask_v[t, pl.ds(st, L)] for st in blks]
            os_ = [off_v[t, pl.ds(st, L)] for st in blks]
            as_ = [d * m for d, m in zip(ds_, ms)]
            cs = [a * o for a, o in zip(as_, os_)]
            for st, a in zip(blks, as_):
                a_v[t, pl.ds(st, L)] = a
            for st, c in zip(blks, cs):
                c_v[t, pl.ds(st, L)] = c
        return 0
    lax.fori_loop(0, _T, fuse_row, 0)

    def grp_body(g, _):
        b = lax.rem(g, 2)

        @pl.when(g >= 2)
        def _drain():
            pltpu.make_async_copy(
                out_hbm.at[pl.ds(base, L)], out_v.at[0], sem).wait()

        tid16 = tid_v[pl.ds(g * L, L)]
        inp16 = inp_v[pl.ds(g * L, L)]
        for k in range(L):
            t = tid16[k]
            s = inp16[k]
            for blks in _chunks(_STARTS, _CH):
                as_ = [a_v[t, pl.ds(st, L)] for st in blks]
                cs = [c_v[t, pl.ds(st, L)] for st in blks]
                outs = [a * s + c for a, c in zip(as_, cs)]
                for st, o in zip(blks, outs):
                    out_v[b, k, pl.ds(st, L)] = o
        pltpu.async_copy(out_v.at[b], out_hbm.at[pl.ds(base + g * L, L)], sem)
        return 0
    lax.fori_loop(0, _NGRP, grp_body, 0)

    for _ in range(2):
        pltpu.make_async_copy(
            out_hbm.at[pl.ds(base, L)], out_v.at[0], sem).wait()


@jax.jit
def _sc_affine(inp1, task_ids, offsets, discrimination, mask):
    kfn = pl.kernel(
        _sc_body,
        out_type=jax.ShapeDtypeStruct((_B, _ML), jnp.float32),
        mesh=plsc.VectorSubcoreMesh(core_axis_name="c", subcore_axis_name="s"),
        scratch_types=[
            pltpu.VMEM((_BPW,), jnp.int32),
            pltpu.VMEM((_BPW,), jnp.float32),
            pltpu.VMEM((_T, _ML), jnp.float32),
            pltpu.VMEM((_T, _ML), jnp.float32),
            pltpu.VMEM((_T, _ML), jnp.float32),
            pltpu.VMEM((_T, _ML), jnp.float32),
            pltpu.VMEM((_T, _ML), jnp.float32),
            pltpu.VMEM((2, L, _ML), jnp.float32),
            pltpu.SemaphoreType.DMA,
        ],
    )
    return kfn(inp1, task_ids, offsets, discrimination, mask)


def kernel(inp, task_ids, offsets, discrimination, mask):
    return _sc_affine(inp.reshape(-1), task_ids, offsets, discrimination, mask)
